# baseline (speedup 1.0000x reference)
import functools

import jax
import jax.numpy as jnp
from jax import lax
from jax.experimental import pallas as pl
from jax.experimental.pallas import tpu as pltpu
from jax.experimental.pallas import tpu_sc as plsc

N = 10000
E = 320000
D = 128
NP = 10240
CHUNK = 40
NC, NS = 2, 16
NW = NC * NS
ROWS_PER_TILE = NP // NS
GCHUNKS = 50
NGROUP = E // (NW * GCHUNKS * CHUNK)


def _node_linear_body(x_ref, wt_ref, b_ref, ax_ref, tdst_ref, tsrc_ref):
    y = jnp.dot(x_ref[...], wt_ref[...], preferred_element_type=jnp.float32)
    y = y + b_ref[...]
    ax_ref[...] = y[:, 0:D]
    tdst_ref[...] = y[:, D:2 * D]
    tsrc_ref[...] = y[:, 2 * D:4 * D]


def _node_linears(xp, wt, bias):
    blk = 1024
    grid = NP // blk
    return pl.pallas_call(
        _node_linear_body,
        grid=(grid,),
        in_specs=[
            pl.BlockSpec((blk, D), lambda i: (i, 0)),
            pl.BlockSpec((D, 4 * D), lambda i: (0, 0)),
            pl.BlockSpec((1, 4 * D), lambda i: (0, 0)),
        ],
        out_specs=[
            pl.BlockSpec((blk, D), lambda i: (i, 0)),
            pl.BlockSpec((blk, D), lambda i: (i, 0)),
            pl.BlockSpec((blk, 2 * D), lambda i: (i, 0)),
        ],
        out_shape=[
            jax.ShapeDtypeStruct((NP, D), jnp.float32),
            jax.ShapeDtypeStruct((NP, D), jnp.float32),
            jax.ShapeDtypeStruct((NP, 2 * D), jnp.float32),
        ],
    )(xp, wt, bias)


def _edge_linear_body(ea_ref, cwt_ref, cb_ref, ce_ref):
    ce_ref[...] = jnp.dot(ea_ref[...], cwt_ref[...],
                          preferred_element_type=jnp.float32) + cb_ref[...]


def _edge_linear(edge_attr, cwt, cb):
    blk = 2560
    grid = E // blk
    return pl.pallas_call(
        _edge_linear_body,
        grid=(grid,),
        in_specs=[
            pl.BlockSpec((blk, D), lambda i: (i, 0)),
            pl.BlockSpec((D, D), lambda i: (0, 0)),
            pl.BlockSpec((1, D), lambda i: (0, 0)),
        ],
        out_specs=pl.BlockSpec((blk, D), lambda i: (i, 0)),
        out_shape=jax.ShapeDtypeStruct((E, D), jnp.float32),
    )(edge_attr, cwt, cb)


DEGR = NP // D


def _sc_edge_body(ce_hbm, src_hbm, dst_hbm, tdst_hbm, tsrc_hbm, zeros_hbm,
                  agg_out,
                  bdst, bsrc, scidx0, scidx1, td, ts, cebuf,
                  aggs, sem_a, sem_b, sem_c, sem_d):
    cid_core = lax.axis_index("c")
    sid = lax.axis_index("s")

    row0 = sid * ROWS_PER_TILE
    pltpu.sync_copy(zeros_hbm.at[pl.ds(row0, ROWS_PER_TILE)],
                    aggs.at[pl.ds(row0, ROWS_PER_TILE)])

    plsc.subcore_barrier()

    wid = sid * NC + cid_core
    gper = GCHUNKS * CHUNK

    def _drain_scatter(slot):
        pltpu.make_async_copy(cebuf.at[slot], aggs.at[pl.ds(0, CHUNK)],
                              sem_d[slot]).wait()

    def _fire_gather(l, slot):
        off = l * CHUNK
        pltpu.async_copy(tdst_hbm.at[bdst.at[pl.ds(off, CHUNK)]],
                         td.at[slot], sem_a[slot])
        pltpu.async_copy(tsrc_hbm.at[bsrc.at[pl.ds(off, CHUNK)]],
                         ts.at[slot], sem_b[slot])

    def _fire_ce(l, slot, gbase):
        base = (gbase + l) * CHUNK
        pltpu.async_copy(ce_hbm.at[pl.ds(base, CHUNK)],
                         cebuf.at[slot], sem_c[slot])

    def _consume(l, slot):
        pltpu.make_async_copy(tdst_hbm.at[pl.ds(0, CHUNK)],
                              td.at[slot], sem_a[slot]).wait()
        pltpu.make_async_copy(tsrc_hbm.at[pl.ds(0, CHUNK)],
                              ts.at[slot], sem_b[slot]).wait()
        pltpu.make_async_copy(ce_hbm.at[pl.ds(0, CHUNK)],
                              cebuf.at[slot], sem_c[slot]).wait()

        def _row(r, rc):
            for c in range(D // 16):
                sl = pl.ds(c * 16, 16)
                z = cebuf[slot, r, sl] + td[slot, r, sl] + ts[slot, r, sl]
                sig = 1.0 / (1.0 + jnp.exp(-z))
                cebuf[slot, r, sl] = sig * ts[slot, r, pl.ds(D + c * 16, 16)]
            return rc

        lax.fori_loop(0, CHUNK, _row, 0)

        off = l * CHUNK
        scidx = scidx0 if slot == 0 else scidx1
        scidx[pl.ds(0, 16)] = bdst[pl.ds(off, 16)]
        scidx[pl.ds(16, 16)] = bdst[pl.ds(off + 16, 16)]
        scidx[pl.ds(CHUNK - 16, 16)] = bdst[pl.ds(off + CHUNK - 16, 16)]

        pltpu.async_copy(cebuf.at[slot], aggs.at[scidx], sem_d[slot],
                         add=True)

    def _group(g, carry):
        gbase = (wid * NGROUP + g) * GCHUNKS
        pltpu.sync_copy(dst_hbm.at[pl.ds(gbase * CHUNK, gper)], bdst)
        pltpu.sync_copy(src_hbm.at[pl.ds(gbase * CHUNK, gper)], bsrc)
        _fire_gather(0, 0)
        _fire_ce(0, 0, gbase)

        def _pair(p, pc):
            l0 = 2 * p
            _fire_gather(l0 + 1, 1)
            _consume(l0, 0)

            @pl.when(p < GCHUNKS // 2 - 1)
            def _():
                _fire_gather(l0 + 2, 0)

            @pl.when(p >= 1)
            def _():
                _drain_scatter(1)

            _fire_ce(l0 + 1, 1, gbase)
            _consume(l0 + 1, 1)

            @pl.when(p < GCHUNKS // 2 - 1)
            def _():
                _drain_scatter(0)
                _fire_ce(l0 + 2, 0, gbase)

            return pc

        lax.fori_loop(0, GCHUNKS // 2, _pair, 0)
        _drain_scatter(0)
        _drain_scatter(1)
        return carry

    lax.fori_loop(0, NGROUP, _group, 0)

    plsc.subcore_barrier()

    pltpu.sync_copy(aggs.at[pl.ds(row0, ROWS_PER_TILE)],
                    agg_out.at[cid_core, pl.ds(row0, ROWS_PER_TILE)])


def _sc_edge(ce, src, dst, tdst, tsrc, zeros):
    mesh = plsc.VectorSubcoreMesh(core_axis_name="c", subcore_axis_name="s",
                                  num_cores=NC, num_subcores=NS)
    f = pl.kernel(
        _sc_edge_body,
        out_type=jax.ShapeDtypeStruct((NC, NP, D), jnp.float32),
        mesh=mesh,
        scratch_types=[
            pltpu.VMEM((GCHUNKS * CHUNK,), jnp.int32),
            pltpu.VMEM((GCHUNKS * CHUNK,), jnp.int32),
            pltpu.VMEM((CHUNK,), jnp.int32),
            pltpu.VMEM((CHUNK,), jnp.int32),
            pltpu.VMEM((2, CHUNK, D), jnp.float32),
            pltpu.VMEM((2, CHUNK, 2 * D), jnp.float32),
            pltpu.VMEM((2, CHUNK, D), jnp.float32),
            pltpu.VMEM_SHARED((NP, D), jnp.float32),
            [pltpu.SemaphoreType.DMA, pltpu.SemaphoreType.DMA],
            [pltpu.SemaphoreType.DMA, pltpu.SemaphoreType.DMA],
            [pltpu.SemaphoreType.DMA, pltpu.SemaphoreType.DMA],
            [pltpu.SemaphoreType.DMA, pltpu.SemaphoreType.DMA],
        ],
    )
    return f(ce, src, dst, tdst, tsrc, zeros)


def _deg_body(dst_ref, deg_ref):
    iq = lax.broadcasted_iota(jnp.int32, (DEGR, 1), 0)
    im = lax.broadcasted_iota(jnp.int32, (D, 1), 0)

    def _col(c, acc):
        row = dst_ref[pl.ds(c, 1), :]
        ohq = (iq == lax.shift_right_logical(row, 7)).astype(jnp.bfloat16)
        ohm = (im == jnp.bitwise_and(row, D - 1)).astype(jnp.bfloat16)
        return acc + lax.dot_general(ohq, ohm, (((1,), (1,)), ((), ())),
                                     preferred_element_type=jnp.float32)

    deg_ref[...] = lax.fori_loop(0, D, _col,
                                 jnp.zeros((DEGR, D), jnp.float32))


def _deg_histogram(dst2d):
    return pl.pallas_call(
        _deg_body,
        out_shape=jax.ShapeDtypeStruct((DEGR, D), jnp.float32),
    )(dst2d)


def _final_body(ax_ref, agg_ref, deg_ref, g_ref, b_ref, h_ref):
    agg = agg_ref[0, 0:N, :] + agg_ref[1, 0:N, :]
    deg = jnp.maximum(deg_ref[0:N, :], 1.0)
    h = ax_ref[0:N, :] + agg * (1.0 / deg)
    mean = jnp.mean(h, axis=0, keepdims=True)
    var = jnp.mean((h - mean) ** 2, axis=0, keepdims=True)
    hn = (h - mean) * jax.lax.rsqrt(var + 1e-5) * g_ref[...] + b_ref[...]
    h_ref[...] = jnp.maximum(hn, 0.0)


def _final(ax, agg2, deg2, gamma, beta):
    return pl.pallas_call(
        _final_body,
        out_shape=jax.ShapeDtypeStruct((N, D), jnp.float32),
    )(ax, agg2, deg2, gamma.reshape(1, D), beta.reshape(1, D))


@jax.jit
def kernel(x, edge_index, edge_attr, A_w, A_b, B_w, B_b, C_w, C_b,
           D_w, D_b, E_w, E_b, bn_gamma, bn_beta):
    xp = jnp.zeros((NP, D), jnp.float32).at[0:N].set(x)
    wt = jnp.concatenate([A_w.T, D_w.T, E_w.T, B_w.T], axis=1)
    bias = jnp.concatenate([A_b, D_b, E_b, B_b]).reshape(1, 4 * D)

    ax, tdst, tsrc = _node_linears(xp, wt, bias)
    ce = _edge_linear(edge_attr, C_w.T, C_b.reshape(1, D))

    src = edge_index[0]
    dst = edge_index[1]
    zeros = jnp.zeros((NP, D), jnp.float32)
    agg2 = _sc_edge(ce, src, dst, tdst, tsrc, zeros)
    deg = _deg_histogram(dst.reshape(D, E // D))

    h = _final(ax, agg2, deg.reshape(NP, 1), bn_gamma, bn_beta)
    return (h, ce)


# speedup vs baseline: 3.3516x; 3.3516x over previous
import functools

import numpy as _np

import jax
import jax.numpy as jnp
from jax import lax
from jax.experimental import pallas as pl
from jax.experimental.pallas import tpu as pltpu
from jax.experimental.pallas import tpu_sc as plsc

N = 10000
E = 320000
D = 128
NP = 10240
CHUNK = 40
NC, NS = 2, 16
NW = NC * NS
ROWS_PER_TILE = NP // NS
GCHUNKS = 50
NGROUP = E // (NW * GCHUNKS * CHUNK)


def _node_linear_body(x_ref, wt_ref, b_ref, ax_ref, tdst_ref, tsrc_ref):
    y = jnp.dot(x_ref[...], wt_ref[...], preferred_element_type=jnp.float32)
    y = y + b_ref[...]
    ax_ref[...] = y[:, 0:D]
    tdst_ref[...] = y[:, D:2 * D]
    tsrc_ref[...] = y[:, 2 * D:4 * D].astype(jnp.bfloat16)


def _node_linears(xp, wt, bias):
    blk = 1024
    grid = NP // blk
    return pl.pallas_call(
        _node_linear_body,
        grid=(grid,),
        in_specs=[
            pl.BlockSpec((blk, D), lambda i: (i, 0)),
            pl.BlockSpec((D, 4 * D), lambda i: (0, 0)),
            pl.BlockSpec((1, 4 * D), lambda i: (0, 0)),
        ],
        out_specs=[
            pl.BlockSpec((blk, D), lambda i: (i, 0)),
            pl.BlockSpec((blk, D), lambda i: (i, 0)),
            pl.BlockSpec((blk, 2 * D), lambda i: (i, 0)),
        ],
        out_shape=[
            jax.ShapeDtypeStruct((NP, D), jnp.float32),
            jax.ShapeDtypeStruct((NP, D), jnp.float32),
            jax.ShapeDtypeStruct((NP, 2 * D), jnp.bfloat16),
        ],
    )(xp, wt, bias)


def _edge_linear_body(ea_ref, cwt_ref, cb_ref, ce_ref):
    ce_ref[...] = jnp.dot(ea_ref[...], cwt_ref[...],
                          preferred_element_type=jnp.float32) + cb_ref[...]


def _edge_linear(edge_attr, cwt, cb):
    blk = 2560
    grid = E // blk
    return pl.pallas_call(
        _edge_linear_body,
        grid=(grid,),
        in_specs=[
            pl.BlockSpec((blk, D), lambda i: (i, 0)),
            pl.BlockSpec((D, D), lambda i: (0, 0)),
            pl.BlockSpec((1, D), lambda i: (0, 0)),
        ],
        out_specs=pl.BlockSpec((blk, D), lambda i: (i, 0)),
        out_shape=jax.ShapeDtypeStruct((E, D), jnp.float32),
    )(edge_attr, cwt, cb)


DEGR = NP // D


def _sc_edge_body(ce_hbm, src_hbm, dst_hbm, tdst_hbm, tsrc_hbm, zeros_hbm,
                  agg_out,
                  bdst, bsrc, scidx, td, ts, cebuf,
                  aggs, sem_a, sem_b, sem_c):
    cid_core = lax.axis_index("c")
    sid = lax.axis_index("s")

    row0 = sid * ROWS_PER_TILE
    pltpu.sync_copy(zeros_hbm.at[pl.ds(row0, ROWS_PER_TILE)],
                    aggs.at[pl.ds(row0, ROWS_PER_TILE)])

    plsc.subcore_barrier()

    wid = sid * NC + cid_core
    gper = GCHUNKS * CHUNK

    def _fire(l, slot, gbase):
        off = l * CHUNK
        base = (gbase + l) * CHUNK
        pltpu.async_copy(tdst_hbm.at[bdst.at[pl.ds(off, CHUNK)]],
                         td.at[slot], sem_a[slot])
        pltpu.async_copy(tsrc_hbm.at[bsrc.at[pl.ds(off, CHUNK)]],
                         ts.at[slot], sem_b[slot])
        pltpu.async_copy(ce_hbm.at[pl.ds(base, CHUNK)],
                         cebuf.at[slot], sem_c[slot])

    def _consume(l, slot):
        pltpu.make_async_copy(tdst_hbm.at[pl.ds(0, CHUNK)],
                              td.at[slot], sem_a[slot]).wait()
        pltpu.make_async_copy(tsrc_hbm.at[pl.ds(0, CHUNK)],
                              ts.at[slot], sem_b[slot]).wait()
        pltpu.make_async_copy(ce_hbm.at[pl.ds(0, CHUNK)],
                              cebuf.at[slot], sem_c[slot]).wait()

        def _unpack(w):
            lo = lax.bitcast_convert_type(lax.shift_left(w, 16), jnp.float32)
            hi = lax.bitcast_convert_type(
                jnp.bitwise_and(w, jnp.int32(-65536)), jnp.float32)
            return lo, hi

        def _row(r, rc):
            for c in range(D // 16):
                sl = pl.ds(c * 16, 16)
                ts_e, ts_b = _unpack(ts[slot, r, sl])
                z = cebuf[slot, r, sl] + td[slot, r, sl] + ts_e
                sig = 1.0 / (1.0 + jnp.exp(-z))
                cebuf[slot, r, sl] = sig * ts_b
            return rc

        lax.fori_loop(0, CHUNK, _row, 0)

        off = l * CHUNK
        scidx[pl.ds(0, 16)] = bdst[pl.ds(off, 16)]
        scidx[pl.ds(16, 16)] = bdst[pl.ds(off + 16, 16)]
        scidx[pl.ds(CHUNK - 16, 16)] = bdst[pl.ds(off + CHUNK - 16, 16)]

        pltpu.sync_copy(cebuf.at[slot], aggs.at[scidx], add=True)

    def _group(g, carry):
        gbase = (wid * NGROUP + g) * GCHUNKS
        pltpu.sync_copy(dst_hbm.at[pl.ds(gbase * CHUNK, gper)], bdst)
        pltpu.sync_copy(src_hbm.at[pl.ds(gbase * CHUNK, gper)], bsrc)
        _fire(0, 0, gbase)

        def _pair(p, pc):
            l0 = 2 * p
            _fire(l0 + 1, 1, gbase)
            _consume(l0, 0)

            @pl.when(p < GCHUNKS // 2 - 1)
            def _():
                _fire(l0 + 2, 0, gbase)

            _consume(l0 + 1, 1)
            return pc

        lax.fori_loop(0, GCHUNKS // 2, _pair, 0)
        return carry

    lax.fori_loop(0, NGROUP, _group, 0)

    plsc.subcore_barrier()

    pltpu.sync_copy(aggs.at[pl.ds(row0, ROWS_PER_TILE)],
                    agg_out.at[cid_core, pl.ds(row0, ROWS_PER_TILE)])


def _sc_edge(ce, src, dst, tdst, tsrc, zeros):
    mesh = plsc.VectorSubcoreMesh(core_axis_name="c", subcore_axis_name="s",
                                  num_cores=NC, num_subcores=NS)
    f = pl.kernel(
        _sc_edge_body,
        out_type=jax.ShapeDtypeStruct((NC, NP, D), jnp.float32),
        mesh=mesh,
        scratch_types=[
            pltpu.VMEM((GCHUNKS * CHUNK,), jnp.int32),
            pltpu.VMEM((GCHUNKS * CHUNK,), jnp.int32),
            pltpu.VMEM((CHUNK,), jnp.int32),
            pltpu.VMEM((2, CHUNK, D), jnp.float32),
            pltpu.VMEM((2, CHUNK, D), jnp.int32),
            pltpu.VMEM((2, CHUNK, D), jnp.float32),
            pltpu.VMEM_SHARED((NP, D), jnp.float32),
            [pltpu.SemaphoreType.DMA, pltpu.SemaphoreType.DMA],
            [pltpu.SemaphoreType.DMA, pltpu.SemaphoreType.DMA],
            [pltpu.SemaphoreType.DMA, pltpu.SemaphoreType.DMA],
        ],
    )
    return f(ce, src, dst, tdst, tsrc, zeros)


def _deg_body(dst_ref, deg_ref):
    iq = lax.broadcasted_iota(jnp.int32, (DEGR, 1), 0)
    im = lax.broadcasted_iota(jnp.int32, (D, 1), 0)

    def _col(c, acc):
        row = dst_ref[pl.ds(c, 1), :]
        ohq = (iq == lax.shift_right_logical(row, 7)).astype(jnp.bfloat16)
        ohm = (im == jnp.bitwise_and(row, D - 1)).astype(jnp.bfloat16)
        return acc + lax.dot_general(ohq, ohm, (((1,), (1,)), ((), ())),
                                     preferred_element_type=jnp.float32)

    deg_ref[...] = lax.fori_loop(0, D, _col,
                                 jnp.zeros((DEGR, D), jnp.float32))


def _deg_histogram(dst2d):
    return pl.pallas_call(
        _deg_body,
        out_shape=jax.ShapeDtypeStruct((DEGR, D), jnp.float32),
    )(dst2d)


def _final_body(ax_ref, agg_ref, deg_ref, g_ref, b_ref, h_ref):
    agg = agg_ref[0, 0:N, :] + agg_ref[1, 0:N, :]
    deg = jnp.maximum(deg_ref[0:N, :], 1.0)
    h = ax_ref[0:N, :] + agg * (1.0 / deg)
    mean = jnp.mean(h, axis=0, keepdims=True)
    var = jnp.mean((h - mean) ** 2, axis=0, keepdims=True)
    hn = (h - mean) * jax.lax.rsqrt(var + 1e-5) * g_ref[...] + b_ref[...]
    h_ref[...] = jnp.maximum(hn, 0.0)


def _final(ax, agg2, deg2, gamma, beta):
    return pl.pallas_call(
        _final_body,
        out_shape=jax.ShapeDtypeStruct((N, D), jnp.float32),
    )(ax, agg2, deg2, gamma.reshape(1, D), beta.reshape(1, D))


@jax.jit
def kernel(x, edge_index, edge_attr, A_w, A_b, B_w, B_b, C_w, C_b,
           D_w, D_b, E_w, E_b, bn_gamma, bn_beta):
    xp = jnp.zeros((NP, D), jnp.float32).at[0:N].set(x)
    k = _np.arange(2 * D)
    perm = 16 * (k // 32) + (k % 32) // 2 + D * (k % 2)
    eb_w = jnp.concatenate([E_w.T, B_w.T], axis=1)[:, perm]
    eb_b = jnp.concatenate([E_b, B_b])[perm]
    wt = jnp.concatenate([A_w.T, D_w.T, eb_w], axis=1)
    bias = jnp.concatenate([A_b, D_b, eb_b]).reshape(1, 4 * D)

    ax, tdst, tsrc = _node_linears(xp, wt, bias)
    ce = _edge_linear(edge_attr, C_w.T, C_b.reshape(1, D))

    src = edge_index[0]
    dst = edge_index[1]
    zeros = jnp.zeros((NP, D), jnp.float32)
    tsrc_p = jax.lax.bitcast_convert_type(tsrc.reshape(NP, D, 2), jnp.int32)
    agg2 = _sc_edge(ce, src, dst, tdst, tsrc_p, zeros)
    deg = _deg_histogram(dst.reshape(D, E // D))

    h = _final(ax, agg2, deg.reshape(NP, 1), bn_gamma, bn_beta)
    return (h, ce)


# speedup vs baseline: 3.3548x; 1.0010x over previous
import functools

import numpy as _np

import jax
import jax.numpy as jnp
from jax import lax
from jax.experimental import pallas as pl
from jax.experimental.pallas import tpu as pltpu
from jax.experimental.pallas import tpu_sc as plsc

N = 10000
E = 320000
D = 128
NP = 10240
CHUNK = 40
NC, NS = 2, 16
NW = NC * NS
ROWS_PER_TILE = NP // NS
GCHUNKS = 50
NGROUP = E // (NW * GCHUNKS * CHUNK)


def _node_linear_body(x_ref, wt_ref, b_ref, ax_ref, tdst_ref, tsrc_ref):
    y = jnp.dot(x_ref[...], wt_ref[...], preferred_element_type=jnp.float32)
    y = y + b_ref[...]
    ax_ref[...] = y[:, 0:D]
    tdst_ref[...] = y[:, D:2 * D]
    tsrc_ref[...] = y[:, 2 * D:4 * D].astype(jnp.bfloat16)


def _node_linears(xp, wt, bias):
    blk = 1024
    grid = NP // blk
    return pl.pallas_call(
        _node_linear_body,
        grid=(grid,),
        in_specs=[
            pl.BlockSpec((blk, D), lambda i: (i, 0)),
            pl.BlockSpec((D, 4 * D), lambda i: (0, 0)),
            pl.BlockSpec((1, 4 * D), lambda i: (0, 0)),
        ],
        out_specs=[
            pl.BlockSpec((blk, D), lambda i: (i, 0)),
            pl.BlockSpec((blk, D), lambda i: (i, 0)),
            pl.BlockSpec((blk, 2 * D), lambda i: (i, 0)),
        ],
        out_shape=[
            jax.ShapeDtypeStruct((NP, D), jnp.float32),
            jax.ShapeDtypeStruct((NP, D), jnp.float32),
            jax.ShapeDtypeStruct((NP, 2 * D), jnp.bfloat16),
        ],
    )(xp, wt, bias)


def _edge_linear_body(ea_ref, cwt_ref, cb_ref, ce_ref):
    ce_ref[...] = jnp.dot(ea_ref[...], cwt_ref[...],
                          preferred_element_type=jnp.float32) + cb_ref[...]


def _edge_linear(edge_attr, cwt, cb):
    blk = 2560
    grid = E // blk
    return pl.pallas_call(
        _edge_linear_body,
        grid=(grid,),
        in_specs=[
            pl.BlockSpec((blk, D), lambda i: (i, 0)),
            pl.BlockSpec((D, D), lambda i: (0, 0)),
            pl.BlockSpec((1, D), lambda i: (0, 0)),
        ],
        out_specs=pl.BlockSpec((blk, D), lambda i: (i, 0)),
        out_shape=jax.ShapeDtypeStruct((E, D), jnp.float32),
    )(edge_attr, cwt, cb)


DEGR = NP // D


def _sc_edge_body(ce_hbm, src_hbm, dst_hbm, tdst_hbm, tsrc_hbm, zeros_hbm,
                  agg_out,
                  bdst, bsrc, scidx, td, ts, cebuf,
                  aggs, sem_a, sem_b, sem_c):
    cid_core = lax.axis_index("c")
    sid = lax.axis_index("s")

    row0 = sid * ROWS_PER_TILE
    pltpu.sync_copy(zeros_hbm.at[pl.ds(row0, ROWS_PER_TILE)],
                    aggs.at[pl.ds(row0, ROWS_PER_TILE)])

    plsc.subcore_barrier()

    wid = sid * NC + cid_core
    gper = GCHUNKS * CHUNK

    def _fire(l, slot, gbase):
        off = l * CHUNK
        base = (gbase + l) * CHUNK
        pltpu.async_copy(tdst_hbm.at[bdst.at[pl.ds(off, CHUNK)]],
                         td.at[slot], sem_a[slot])
        pltpu.async_copy(tsrc_hbm.at[bsrc.at[pl.ds(off, CHUNK)]],
                         ts.at[slot], sem_b[slot])
        pltpu.async_copy(ce_hbm.at[pl.ds(base, CHUNK)],
                         cebuf.at[slot], sem_c[slot])

    def _consume(l, slot):
        pltpu.make_async_copy(tdst_hbm.at[pl.ds(0, CHUNK)],
                              td.at[slot], sem_a[slot]).wait()
        pltpu.make_async_copy(tsrc_hbm.at[pl.ds(0, CHUNK)],
                              ts.at[slot], sem_b[slot]).wait()
        pltpu.make_async_copy(ce_hbm.at[pl.ds(0, CHUNK)],
                              cebuf.at[slot], sem_c[slot]).wait()

        def _unpack(w):
            lo = lax.bitcast_convert_type(lax.shift_left(w, 16), jnp.float32)
            hi = lax.bitcast_convert_type(
                jnp.bitwise_and(w, jnp.int32(-65536)), jnp.float32)
            return lo, hi

        @plsc.parallel_loop(0, CHUNK, unroll=2)
        def _row(r):
            for c in range(D // 16):
                sl = pl.ds(c * 16, 16)
                ts_e, ts_b = _unpack(ts[slot, r, sl])
                z = cebuf[slot, r, sl] + td[slot, r, sl] + ts_e
                sig = 1.0 / (1.0 + jnp.exp(-z))
                cebuf[slot, r, sl] = sig * ts_b

        off = l * CHUNK
        scidx[pl.ds(0, 16)] = bdst[pl.ds(off, 16)]
        scidx[pl.ds(16, 16)] = bdst[pl.ds(off + 16, 16)]
        scidx[pl.ds(CHUNK - 16, 16)] = bdst[pl.ds(off + CHUNK - 16, 16)]

        pltpu.sync_copy(cebuf.at[slot], aggs.at[scidx], add=True)

    def _group(g, carry):
        gbase = (wid * NGROUP + g) * GCHUNKS
        pltpu.sync_copy(dst_hbm.at[pl.ds(gbase * CHUNK, gper)], bdst)
        pltpu.sync_copy(src_hbm.at[pl.ds(gbase * CHUNK, gper)], bsrc)
        _fire(0, 0, gbase)

        def _pair(p, pc):
            l0 = 2 * p
            _fire(l0 + 1, 1, gbase)
            _consume(l0, 0)

            @pl.when(p < GCHUNKS // 2 - 1)
            def _():
                _fire(l0 + 2, 0, gbase)

            _consume(l0 + 1, 1)
            return pc

        lax.fori_loop(0, GCHUNKS // 2, _pair, 0)
        return carry

    lax.fori_loop(0, NGROUP, _group, 0)

    plsc.subcore_barrier()

    pltpu.sync_copy(aggs.at[pl.ds(row0, ROWS_PER_TILE)],
                    agg_out.at[cid_core, pl.ds(row0, ROWS_PER_TILE)])


def _sc_edge(ce, src, dst, tdst, tsrc, zeros):
    mesh = plsc.VectorSubcoreMesh(core_axis_name="c", subcore_axis_name="s",
                                  num_cores=NC, num_subcores=NS)
    f = pl.kernel(
        _sc_edge_body,
        out_type=jax.ShapeDtypeStruct((NC, NP, D), jnp.float32),
        mesh=mesh,
        scratch_types=[
            pltpu.VMEM((GCHUNKS * CHUNK,), jnp.int32),
            pltpu.VMEM((GCHUNKS * CHUNK,), jnp.int32),
            pltpu.VMEM((CHUNK,), jnp.int32),
            pltpu.VMEM((2, CHUNK, D), jnp.float32),
            pltpu.VMEM((2, CHUNK, D), jnp.int32),
            pltpu.VMEM((2, CHUNK, D), jnp.float32),
            pltpu.VMEM_SHARED((NP, D), jnp.float32),
            [pltpu.SemaphoreType.DMA, pltpu.SemaphoreType.DMA],
            [pltpu.SemaphoreType.DMA, pltpu.SemaphoreType.DMA],
            [pltpu.SemaphoreType.DMA, pltpu.SemaphoreType.DMA],
        ],
    )
    return f(ce, src, dst, tdst, tsrc, zeros)


def _deg_body(dst_ref, deg_ref):
    iq = lax.broadcasted_iota(jnp.int32, (DEGR, 1), 0)
    im = lax.broadcasted_iota(jnp.int32, (D, 1), 0)

    def _col(c, acc):
        row = dst_ref[pl.ds(c, 1), :]
        ohq = (iq == lax.shift_right_logical(row, 7)).astype(jnp.bfloat16)
        ohm = (im == jnp.bitwise_and(row, D - 1)).astype(jnp.bfloat16)
        return acc + lax.dot_general(ohq, ohm, (((1,), (1,)), ((), ())),
                                     preferred_element_type=jnp.float32)

    deg_ref[...] = lax.fori_loop(0, D, _col,
                                 jnp.zeros((DEGR, D), jnp.float32))


def _deg_histogram(dst2d):
    return pl.pallas_call(
        _deg_body,
        out_shape=jax.ShapeDtypeStruct((DEGR, D), jnp.float32),
    )(dst2d)


def _final_body(ax_ref, agg_ref, deg_ref, g_ref, b_ref, h_ref):
    agg = agg_ref[0, 0:N, :] + agg_ref[1, 0:N, :]
    deg = jnp.maximum(deg_ref[0:N, :], 1.0)
    h = ax_ref[0:N, :] + agg * (1.0 / deg)
    mean = jnp.mean(h, axis=0, keepdims=True)
    var = jnp.mean((h - mean) ** 2, axis=0, keepdims=True)
    hn = (h - mean) * jax.lax.rsqrt(var + 1e-5) * g_ref[...] + b_ref[...]
    h_ref[...] = jnp.maximum(hn, 0.0)


def _final(ax, agg2, deg2, gamma, beta):
    return pl.pallas_call(
        _final_body,
        out_shape=jax.ShapeDtypeStruct((N, D), jnp.float32),
    )(ax, agg2, deg2, gamma.reshape(1, D), beta.reshape(1, D))


@jax.jit
def kernel(x, edge_index, edge_attr, A_w, A_b, B_w, B_b, C_w, C_b,
           D_w, D_b, E_w, E_b, bn_gamma, bn_beta):
    xp = jnp.zeros((NP, D), jnp.float32).at[0:N].set(x)
    k = _np.arange(2 * D)
    perm = 16 * (k // 32) + (k % 32) // 2 + D * (k % 2)
    eb_w = jnp.concatenate([E_w.T, B_w.T], axis=1)[:, perm]
    eb_b = jnp.concatenate([E_b, B_b])[perm]
    wt = jnp.concatenate([A_w.T, D_w.T, eb_w], axis=1)
    bias = jnp.concatenate([A_b, D_b, eb_b]).reshape(1, 4 * D)

    ax, tdst, tsrc = _node_linears(xp, wt, bias)
    ce = _edge_linear(edge_attr, C_w.T, C_b.reshape(1, D))

    src = edge_index[0]
    dst = edge_index[1]
    zeros = jnp.zeros((NP, D), jnp.float32)
    tsrc_p = jax.lax.bitcast_convert_type(tsrc.reshape(NP, D, 2), jnp.int32)
    agg2 = _sc_edge(ce, src, dst, tdst, tsrc_p, zeros)
    deg = _deg_histogram(dst.reshape(D, E // D))

    h = _final(ax, agg2, deg.reshape(NP, 1), bn_gamma, bn_beta)
    return (h, ce)
